# Initial kernel scaffold; baseline (speedup 1.0000x reference)
import jax
import jax.numpy as jnp
from jax.experimental import pallas as pl


def kernel(config, edge_index, batch, params):
    raise NotImplementedError("write your pallas kernel here")


# speedup vs baseline: 2.6345x; 2.6345x over previous
import functools
import jax
import jax.numpy as jnp
from jax import lax
from jax.experimental import pallas as pl
from jax.experimental.pallas import tpu as pltpu
from jax.experimental.pallas import tpu_sc as plsc

N = 100000
E = 1600000
G = 16
H = 32
HALF = N // 2
E_PAD = 1601536
DEG_PAD = 102400
CHUNK = 128
TE_16 = E_PAD // 16
TE_32 = E_PAD // 32
NC_16 = TE_16 // CHUNK
NC_32 = TE_32 // CHUNK
AGG_ROWS = 50176
ZROWS = 3136

_mesh = plsc.VectorSubcoreMesh(core_axis_name="c", subcore_axis_name="s")
_SC_PARAMS = pltpu.CompilerParams(use_tc_tiling_on_sc=False)


@functools.partial(
    pl.kernel, mesh=_mesh, compiler_params=_SC_PARAMS,
    out_type=jax.ShapeDtypeStruct((DEG_PAD,), jnp.float32),
    scratch_types=[
        pltpu.VMEM((CHUNK,), jnp.int32),
        pltpu.VMEM((CHUNK,), jnp.float32),
        pltpu.VMEM_SHARED((DEG_PAD,), jnp.float32),
    ],
)
def _sc_degree(col_hbm, zeros_hbm, ones_hbm, deg_hbm, colv, onev, deg_sh):
    cid = lax.axis_index("c")
    sid = lax.axis_index("s")

    @pl.when(cid == 0)
    def _():
        pltpu.sync_copy(zeros_hbm, deg_sh.at[pl.ds(sid * 6400, 6400)])
        plsc.subcore_barrier()
        pltpu.sync_copy(ones_hbm, onev)

        def body(g, _):
            base = sid * TE_16 + g * CHUNK
            pltpu.sync_copy(col_hbm.at[pl.ds(base, CHUNK)], colv)
            pltpu.sync_copy(onev, deg_sh.at[colv], add=True)
            return _

        lax.fori_loop(0, NC_16, body, None)
        plsc.subcore_barrier()
        pltpu.sync_copy(deg_sh.at[pl.ds(sid * 6400, 6400)],
                        deg_hbm.at[pl.ds(sid * 6400, 6400)])


@functools.partial(
    pl.kernel, mesh=_mesh, compiler_params=_SC_PARAMS,
    out_type=(jax.ShapeDtypeStruct((E_PAD,), jnp.float32),
              jax.ShapeDtypeStruct((E_PAD,), jnp.float32)),
    scratch_types=[
        pltpu.VMEM((CHUNK,), jnp.int32),
        pltpu.VMEM((CHUNK,), jnp.float32),
        pltpu.SemaphoreType.DMA,
    ],
)
def _sc_gather_deg(row_hbm, col_hbm, deg_hbm, dr_hbm, dc_hbm, idxv, valv, sem):
    cid = lax.axis_index("c")
    sid = lax.axis_index("s")
    wid = sid * 2 + cid

    def body(g, _):
        base = wid * TE_32 + g * CHUNK
        sl = pl.ds(base, CHUNK)
        pltpu.sync_copy(row_hbm.at[sl], idxv)
        pltpu.async_copy(deg_hbm.at[idxv], valv, sem).wait()
        pltpu.sync_copy(valv, dr_hbm.at[sl])
        pltpu.sync_copy(col_hbm.at[sl], idxv)
        pltpu.async_copy(deg_hbm.at[idxv], valv, sem).wait()
        pltpu.sync_copy(valv, dc_hbm.at[sl])
        return _

    lax.fori_loop(0, NC_32, body, None)


@functools.partial(
    pl.kernel, mesh=_mesh, compiler_params=_SC_PARAMS,
    out_type=jax.ShapeDtypeStruct((N, H), jnp.float32),
    scratch_types=[
        pltpu.VMEM((CHUNK,), jnp.int32),
        pltpu.VMEM((CHUNK,), jnp.int32),
        pltpu.VMEM((CHUNK,), jnp.int32),
        pltpu.VMEM((CHUNK, H), jnp.float32),
        pltpu.VMEM((CHUNK, H), jnp.float32),
        pltpu.VMEM_SHARED((AGG_ROWS, H), jnp.float32),
        pltpu.SemaphoreType.DMA,
    ],
)
def _sc_message(row_hbm, col_hbm, x_hbm, ea_hbm, zrows_hbm, agg_hbm,
                rowv, colv, lcolv, xg, eav, agg_sh, sem):
    cid = lax.axis_index("c")
    sid = lax.axis_index("s")
    nbase = cid * HALF

    pltpu.sync_copy(zrows_hbm, agg_sh.at[pl.ds(sid * ZROWS, ZROWS)])
    plsc.subcore_barrier()

    def body(g, _):
        base = sid * TE_16 + g * CHUNK
        sl = pl.ds(base, CHUNK)
        pltpu.sync_copy(row_hbm.at[sl], rowv)
        pltpu.sync_copy(col_hbm.at[sl], colv)
        pltpu.async_copy(x_hbm.at[rowv], xg, sem).wait()
        pltpu.sync_copy(ea_hbm.at[sl], eav)

        def msg_body(r, _2):
            a = xg[r, pl.ds(0, 16)] + eav[r, pl.ds(0, 16)]
            xg[r, pl.ds(0, 16)] = jnp.maximum(a, 0.0)
            b = xg[r, pl.ds(16, 16)] + eav[r, pl.ds(16, 16)]
            xg[r, pl.ds(16, 16)] = jnp.maximum(b, 0.0)
            return _2

        lax.fori_loop(0, CHUNK, msg_body, None)

        def idx_body(j, _2):
            c16 = colv[pl.ds(j * 16, 16)]
            inr = (c16 >= nbase) & (c16 < nbase + HALF)
            lcolv[pl.ds(j * 16, 16)] = jnp.where(inr, c16 - nbase, HALF)
            return _2

        lax.fori_loop(0, 8, idx_body, None)
        pltpu.sync_copy(xg, agg_sh.at[lcolv], add=True)
        return _

    lax.fori_loop(0, NC_16, body, None)
    plsc.subcore_barrier()

    @pl.when(sid < 15)
    def _():
        pltpu.sync_copy(agg_sh.at[pl.ds(sid * 3128, 3128)],
                        agg_hbm.at[pl.ds(nbase + sid * 3128, 3128)])

    @pl.when(sid == 15)
    def _():
        pltpu.sync_copy(agg_sh.at[pl.ds(46920, 3080)],
                        agg_hbm.at[pl.ds(nbase + 46920, 3080)])


BN = 2000
NGRID = N // BN
BE = 4096
EGRID = E_PAD // BE

_DOT = dict(precision=lax.Precision.HIGHEST, preferred_element_type=jnp.float32)


def _onehot(batch_blk):
    iota = lax.broadcasted_iota(jnp.int32, (1, G), 1)
    return (batch_blk == iota).astype(jnp.float32)


def _mlp2_tc(x, w1, b1, w2, b2):
    h = jnp.maximum(jnp.dot(x, w1, **_DOT) + b1, 0.0)
    return jnp.dot(h, w2, **_DOT) + b2


def _ln_tc(x, g, b):
    mu = jnp.mean(x, axis=-1, keepdims=True)
    var = jnp.mean((x - mu) ** 2, axis=-1, keepdims=True)
    return (x - mu) / jnp.sqrt(var + 1e-5) * g + b


def _full(shape):
    return pl.BlockSpec(shape, lambda i: (0, 0))


def _node_init_body(cfg_ref, b_ref, w1, b1, w2, b2, x_ref, gc_ref, isr_ref):
    i = pl.program_id(0)
    x_ref[...] = _mlp2_tc(cfg_ref[...], w1[...], b1[...], w2[...], b2[...])
    oh = _onehot(b_ref[...])

    @pl.when(i == 0)
    def _():
        gc_ref[...] = jnp.zeros_like(gc_ref)

    gc_ref[...] += jnp.sum(oh, axis=0).reshape(G, 1)

    @pl.when(i == NGRID - 1)
    def _():
        gc = gc_ref[...]
        isr_ref[...] = jnp.where(gc > 0, 1.0 / jnp.sqrt(gc), 0.0)


def _tc_node_init(cfg, batch2, p):
    return pl.pallas_call(
        _node_init_body,
        grid=(NGRID,),
        in_specs=[pl.BlockSpec((BN, 1), lambda i: (i, 0)),
                  pl.BlockSpec((BN, 1), lambda i: (i, 0)),
                  _full((1, H)), _full((1, H)), _full((H, H)), _full((1, H))],
        out_specs=[pl.BlockSpec((BN, H), lambda i: (i, 0)),
                   _full((G, 1)), _full((G, 1))],
        out_shape=[jax.ShapeDtypeStruct((N, H), jnp.float32),
                   jax.ShapeDtypeStruct((G, 1), jnp.float32),
                   jax.ShapeDtypeStruct((G, 1), jnp.float32)],
    )(cfg, batch2, p["w1"], p["b1"].reshape(1, H), p["w2"], p["b2"].reshape(1, H))


def _edge_mlp_body(dr_ref, dc_ref, w1, b1, w2, b2, ea_ref):
    w = w1[...]
    h = jnp.dot(dr_ref[...], w[0:1], **_DOT) + jnp.dot(dc_ref[...], w[1:2], **_DOT)
    h = jnp.maximum(h + b1[...], 0.0)
    ea_ref[...] = jnp.dot(h, w2[...], **_DOT) + b2[...]


def _tc_edge_mlp(dr2, dc2, p):
    return pl.pallas_call(
        _edge_mlp_body,
        grid=(EGRID,),
        in_specs=[pl.BlockSpec((BE, 1), lambda i: (i, 0)),
                  pl.BlockSpec((BE, 1), lambda i: (i, 0)),
                  _full((2, H)), _full((1, H)), _full((H, H)), _full((1, H))],
        out_specs=pl.BlockSpec((BE, H), lambda i: (i, 0)),
        out_shape=jax.ShapeDtypeStruct((E_PAD, H), jnp.float32),
    )(dr2, dc2, p["w1"], p["b1"].reshape(1, H), p["w2"], p["b2"].reshape(1, H))


def _vn_update_body(x_ref, b_ref, vnin, w1, b1, ln1g, ln1b, w2, b2, ln2g, ln2b,
                    vn_ref):
    i = pl.program_id(0)

    @pl.when(i == 0)
    def _():
        vn_ref[...] = jnp.zeros_like(vn_ref)

    oh = _onehot(b_ref[...])
    vn_ref[...] += jnp.dot(oh.T, x_ref[...], **_DOT)

    @pl.when(i == NGRID - 1)
    def _():
        vtemp = vn_ref[...] + vnin[...]
        t = jnp.dot(vtemp, w1[...], **_DOT) + b1[...]
        t = jnp.maximum(_ln_tc(t, ln1g[...], ln1b[...]), 0.0)
        t = jnp.dot(t, w2[...], **_DOT) + b2[...]
        vn_ref[...] = jnp.maximum(_ln_tc(t, ln2g[...], ln2b[...]), 0.0)


def _tc_vn_update(x, batch2, vn_in, vp):
    return pl.pallas_call(
        _vn_update_body,
        grid=(NGRID,),
        in_specs=[pl.BlockSpec((BN, H), lambda i: (i, 0)),
                  pl.BlockSpec((BN, 1), lambda i: (i, 0)),
                  _full((G, H)), _full((H, H)), _full((1, H)), _full((1, H)),
                  _full((1, H)), _full((H, H)), _full((1, H)), _full((1, H)),
                  _full((1, H))],
        out_specs=_full((G, H)),
        out_shape=jax.ShapeDtypeStruct((G, H), jnp.float32),
    )(x, batch2, vn_in, vp["w1"], vp["b1"].reshape(1, H),
      vp["ln1_g"].reshape(1, H), vp["ln1_b"].reshape(1, H),
      vp["w2"], vp["b2"].reshape(1, H),
      vp["ln2_g"].reshape(1, H), vp["ln2_b"].reshape(1, H))


def _node_update_body(x_ref, agg_ref, b_ref, vn, isr, eps, w1, b1, w2, b2,
                      lng, lnb, out_ref):
    x = x_ref[...]
    u = (1.0 + eps[0, 0]) * x + agg_ref[...]
    z = _mlp2_tc(u, w1[...], b1[...], w2[...], b2[...])
    oh = _onehot(b_ref[...])
    z = z * jnp.dot(oh, isr[...], **_DOT)
    z = jnp.maximum(_ln_tc(z, lng[...], lnb[...]), 0.0)
    out_ref[...] = z + x + jnp.dot(oh, vn[...], **_DOT)


def _tc_node_update(x, agg, batch2, vn, isr, eps2, cp, lp):
    return pl.pallas_call(
        _node_update_body,
        grid=(NGRID,),
        in_specs=[pl.BlockSpec((BN, H), lambda i: (i, 0)),
                  pl.BlockSpec((BN, H), lambda i: (i, 0)),
                  pl.BlockSpec((BN, 1), lambda i: (i, 0)),
                  _full((G, H)), _full((G, 1)), _full((1, 1)),
                  _full((H, H)), _full((1, H)), _full((H, H)), _full((1, H)),
                  _full((1, H)), _full((1, H))],
        out_specs=pl.BlockSpec((BN, H), lambda i: (i, 0)),
        out_shape=jax.ShapeDtypeStruct((N, H), jnp.float32),
    )(x, agg, batch2, vn, isr, eps2,
      cp["w1"], cp["b1"].reshape(1, H), cp["w2"], cp["b2"].reshape(1, H),
      lp["g"].reshape(1, H), lp["b"].reshape(1, H))


_NEG = -3.4e38


def _gate_body(x_ref, b_ref, w1, b1, bng, bnb, w2, b2, g_ref, gmax_ref):
    i = pl.program_id(0)
    t = jnp.dot(x_ref[...], w1[...], **_DOT) + b1[...]
    t = t / jnp.sqrt(1.0 + 1e-5) * bng[...] + bnb[...]
    t = jnp.maximum(t, 0.0)
    gn = jnp.dot(t, w2[...], **_DOT) + b2[...]
    g_ref[...] = gn

    @pl.when(i == 0)
    def _():
        gmax_ref[...] = jnp.full_like(gmax_ref, _NEG)

    oh = _onehot(b_ref[...])
    masked = jnp.where(oh > 0, gn, _NEG)
    gmax_ref[...] = jnp.maximum(gmax_ref[...], jnp.max(masked, axis=0).reshape(G, 1))


def _tc_gate(x, batch2, gp):
    return pl.pallas_call(
        _gate_body,
        grid=(NGRID,),
        in_specs=[pl.BlockSpec((BN, H), lambda i: (i, 0)),
                  pl.BlockSpec((BN, 1), lambda i: (i, 0)),
                  _full((H, 2 * H)), _full((1, 2 * H)), _full((1, 2 * H)),
                  _full((1, 2 * H)), _full((2 * H, 1)), _full((1, 1))],
        out_specs=[pl.BlockSpec((BN, 1), lambda i: (i, 0)), _full((G, 1))],
        out_shape=[jax.ShapeDtypeStruct((N, 1), jnp.float32),
                   jax.ShapeDtypeStruct((G, 1), jnp.float32)],
    )(x, batch2, gp["w1"], gp["b1"].reshape(1, 2 * H), gp["bn_g"].reshape(1, 2 * H),
      gp["bn_b"].reshape(1, 2 * H), gp["w2"], gp["b2"].reshape(1, 1))


def _pool_body(x_ref, g_ref, b_ref, gmax, w1, b1, w2, b2,
               s1_ref, s0_ref, out_ref):
    i = pl.program_id(0)

    @pl.when(i == 0)
    def _():
        s1_ref[...] = jnp.zeros_like(s1_ref)
        s0_ref[...] = jnp.zeros_like(s0_ref)
        out_ref[...] = jnp.zeros_like(out_ref)

    oh = _onehot(b_ref[...])
    e = jnp.exp(g_ref[...] - jnp.dot(oh, gmax[...], **_DOT))
    s1_ref[...] += jnp.dot(oh.T, e * x_ref[...], **_DOT)
    s0_ref[...] += jnp.dot(oh.T, e, **_DOT)

    @pl.when(i == NGRID - 1)
    def _():
        s0 = s0_ref[...]
        pool = jnp.where(s0 > 0, s1_ref[...] / jnp.where(s0 > 0, s0, 1.0), 0.0)
        t = _mlp2_tc(pool, w1[...], b1[...], w2[...], b2[...])
        out_ref[...] = 1.0 / (1.0 + jnp.exp(-t))


def _tc_pool_final(x, g, batch2, gmax, fp):
    _, _, out = pl.pallas_call(
        _pool_body,
        grid=(NGRID,),
        in_specs=[pl.BlockSpec((BN, H), lambda i: (i, 0)),
                  pl.BlockSpec((BN, 1), lambda i: (i, 0)),
                  pl.BlockSpec((BN, 1), lambda i: (i, 0)),
                  _full((G, 1)), _full((H, H)), _full((1, H)), _full((H, 1)),
                  _full((1, 1))],
        out_specs=[_full((G, H)), _full((G, 1)), _full((G, 1))],
        out_shape=[jax.ShapeDtypeStruct((G, H), jnp.float32),
                   jax.ShapeDtypeStruct((G, 1), jnp.float32),
                   jax.ShapeDtypeStruct((G, 1), jnp.float32)],
    )(x, g, batch2, gmax, fp["w1"], fp["b1"].reshape(1, H), fp["w2"],
      fp["b2"].reshape(1, 1))
    return out


def kernel(config, edge_index, batch, params):
    pad = E_PAD - E
    row = jnp.concatenate([edge_index[0], jnp.zeros((pad,), jnp.int32)])
    col = jnp.concatenate([edge_index[1], jnp.full((pad,), N, jnp.int32)])
    cfg2 = config.astype(jnp.float32).reshape(N, 1)
    batch2 = batch.reshape(N, 1)
    zeros_deg = jnp.zeros((6400,), jnp.float32)
    ones128 = jnp.ones((CHUNK,), jnp.float32)
    zrows = jnp.zeros((ZROWS, H), jnp.float32)

    deg = _sc_degree(col, zeros_deg, ones128)
    x, _gc, isr = _tc_node_init(cfg2, batch2, params["node_mlp"])
    dr, dc = _sc_gather_deg(row, col, deg)
    ea = _tc_edge_mlp(dr.reshape(E_PAD, 1), dc.reshape(E_PAD, 1),
                      params["edge_mlp"])

    vn = jnp.broadcast_to(params["vn_emb"][0], (G, H))
    for i in range(3):
        agg = _sc_message(row, col, x, ea, zrows)
        vn = _tc_vn_update(x, batch2, vn, params["vn_mlps"][i])
        eps2 = params["convs"][i]["eps"].reshape(1, 1)
        x = _tc_node_update(x, agg, batch2, vn, isr, eps2,
                            params["convs"][i]["nn"], params["lns"][i])

    g, gmax = _tc_gate(x, batch2, params["gate"])
    return _tc_pool_final(x, g, batch2, gmax, params["final_mlp"])


# speedup vs baseline: 3.8489x; 1.4610x over previous
import functools
import jax
import jax.numpy as jnp
from jax import lax
from jax.experimental import pallas as pl
from jax.experimental.pallas import tpu as pltpu
from jax.experimental.pallas import tpu_sc as plsc

N = 100000
E = 1600000
G = 16
H = 32
HALF = N // 2
E_PAD = 1601536
DEG_PAD = 102400
CHUNK = 128
TE_16 = E_PAD // 16
TE_32 = E_PAD // 32
NC_16 = TE_16 // CHUNK
NC_32 = TE_32 // CHUNK
AGG_ROWS = 50176
ZROWS = 3136

_mesh = plsc.VectorSubcoreMesh(core_axis_name="c", subcore_axis_name="s")
_SC_PARAMS = pltpu.CompilerParams(use_tc_tiling_on_sc=False)


@functools.partial(
    pl.kernel, mesh=_mesh, compiler_params=_SC_PARAMS,
    out_type=jax.ShapeDtypeStruct((DEG_PAD,), jnp.float32),
    scratch_types=[
        pltpu.VMEM((CHUNK,), jnp.int32),
        pltpu.VMEM((CHUNK,), jnp.float32),
        pltpu.VMEM_SHARED((DEG_PAD,), jnp.float32),
    ],
)
def _sc_degree(col_hbm, zeros_hbm, ones_hbm, deg_hbm, colv, onev, deg_sh):
    cid = lax.axis_index("c")
    sid = lax.axis_index("s")

    @pl.when(cid == 0)
    def _():
        pltpu.sync_copy(zeros_hbm, deg_sh.at[pl.ds(sid * 6400, 6400)])
        plsc.subcore_barrier()
        pltpu.sync_copy(ones_hbm, onev)

        def body(g, _):
            base = sid * TE_16 + g * CHUNK
            pltpu.sync_copy(col_hbm.at[pl.ds(base, CHUNK)], colv)
            pltpu.sync_copy(onev, deg_sh.at[colv], add=True)
            return _

        lax.fori_loop(0, NC_16, body, None)
        plsc.subcore_barrier()
        pltpu.sync_copy(deg_sh.at[pl.ds(sid * 6400, 6400)],
                        deg_hbm.at[pl.ds(sid * 6400, 6400)])


@functools.partial(
    pl.kernel, mesh=_mesh, compiler_params=_SC_PARAMS,
    out_type=(jax.ShapeDtypeStruct((E_PAD,), jnp.float32),
              jax.ShapeDtypeStruct((E_PAD,), jnp.float32)),
    scratch_types=[
        pltpu.VMEM((CHUNK,), jnp.int32),
        pltpu.VMEM((CHUNK,), jnp.float32),
        pltpu.SemaphoreType.DMA,
    ],
)
def _sc_gather_deg(row_hbm, col_hbm, deg_hbm, dr_hbm, dc_hbm, idxv, valv, sem):
    cid = lax.axis_index("c")
    sid = lax.axis_index("s")
    wid = sid * 2 + cid

    def body(g, _):
        base = wid * TE_32 + g * CHUNK
        sl = pl.ds(base, CHUNK)
        pltpu.sync_copy(row_hbm.at[sl], idxv)
        pltpu.async_copy(deg_hbm.at[idxv], valv, sem).wait()
        pltpu.sync_copy(valv, dr_hbm.at[sl])
        pltpu.sync_copy(col_hbm.at[sl], idxv)
        pltpu.async_copy(deg_hbm.at[idxv], valv, sem).wait()
        pltpu.sync_copy(valv, dc_hbm.at[sl])
        return _

    lax.fori_loop(0, NC_32, body, None)


@functools.partial(
    pl.kernel, mesh=_mesh, compiler_params=_SC_PARAMS,
    out_type=jax.ShapeDtypeStruct((N, H), jnp.float32),
    scratch_types=[
        [pltpu.VMEM((CHUNK,), jnp.int32)] * 2,
        [pltpu.VMEM((CHUNK,), jnp.int32)] * 2,
        [pltpu.VMEM((CHUNK,), jnp.int32)] * 2,
        [pltpu.VMEM((CHUNK, H), jnp.float32)] * 2,
        [pltpu.VMEM((CHUNK, H), jnp.float32)] * 2,
        pltpu.VMEM_SHARED((AGG_ROWS, H), jnp.float32),
        [pltpu.SemaphoreType.DMA] * 2,
        [pltpu.SemaphoreType.DMA] * 2,
        [pltpu.SemaphoreType.DMA] * 2,
        [pltpu.SemaphoreType.DMA] * 2,
    ],
)
def _sc_message(row_hbm, col_hbm, x_hbm, ea_hbm, zrows_hbm, agg_hbm,
                rowv, colv, lcolv, xg, eav, agg_sh, sr, sc_, sx, se):
    cid = lax.axis_index("c")
    sid = lax.axis_index("s")
    nbase = cid * HALF
    ebase = sid * TE_16

    pltpu.sync_copy(zrows_hbm, agg_sh.at[pl.ds(sid * ZROWS, ZROWS)])
    plsc.subcore_barrier()

    def sl(g):
        return pl.ds(ebase + g * CHUNK, CHUNK)

    def start_idx(g, b):
        pltpu.async_copy(row_hbm.at[sl(g)], rowv[b], sr[b])
        pltpu.async_copy(col_hbm.at[sl(g)], colv[b], sc_[b])

    def wait_idx(g, b):
        pltpu.make_async_copy(row_hbm.at[sl(g)], rowv[b], sr[b]).wait()
        pltpu.make_async_copy(col_hbm.at[sl(g)], colv[b], sc_[b]).wait()

    def mk_lcol(b):
        def idx_body(j, _2):
            c16 = colv[b][pl.ds(j * 16, 16)]
            inr = (c16 >= nbase) & (c16 < nbase + HALF)
            lcolv[b][pl.ds(j * 16, 16)] = jnp.where(inr, c16 - nbase, HALF)
            return _2
        lax.fori_loop(0, 8, idx_body, None)

    def start_data(g, b):
        pltpu.async_copy(x_hbm.at[rowv[b]], xg[b], sx[b])
        pltpu.async_copy(ea_hbm.at[sl(g)], eav[b], se[b])

    def wait_data(g, b):
        pltpu.make_async_copy(x_hbm.at[rowv[b]], xg[b], sx[b]).wait()
        pltpu.make_async_copy(ea_hbm.at[sl(g)], eav[b], se[b]).wait()

    def compute_scatter(b):
        def msg_body(r4, _2):
            r = r4 * 4
            for k in range(4):
                a = xg[b][r + k, pl.ds(0, 16)] + eav[b][r + k, pl.ds(0, 16)]
                xg[b][r + k, pl.ds(0, 16)] = jnp.maximum(a, 0.0)
                bb = xg[b][r + k, pl.ds(16, 16)] + eav[b][r + k, pl.ds(16, 16)]
                xg[b][r + k, pl.ds(16, 16)] = jnp.maximum(bb, 0.0)
            return _2
        lax.fori_loop(0, CHUNK // 4, msg_body, None)
        pltpu.sync_copy(xg[b], agg_sh.at[lcolv[b]], add=True)

    def emit(g, p, q, prefetch_idx):
        wait_idx(g + 1, q)
        mk_lcol(q)
        start_data(g + 1, q)
        wait_data(g, p)
        if prefetch_idx:
            start_idx(g + 2, p)
        compute_scatter(p)

    start_idx(0, 0)
    start_idx(1, 1)
    wait_idx(0, 0)
    mk_lcol(0)
    start_data(0, 0)

    def pair_body(i, _):
        g = i * 2
        emit(g, 0, 1, True)
        emit(g + 1, 1, 0, True)
        return _

    lax.fori_loop(0, (NC_16 - 2) // 2, pair_body, None)
    emit(NC_16 - 2, 0, 1, False)
    wait_data(NC_16 - 1, 1)
    compute_scatter(1)
    plsc.subcore_barrier()

    @pl.when(sid < 15)
    def _():
        pltpu.sync_copy(agg_sh.at[pl.ds(sid * 3128, 3128)],
                        agg_hbm.at[pl.ds(nbase + sid * 3128, 3128)])

    @pl.when(sid == 15)
    def _():
        pltpu.sync_copy(agg_sh.at[pl.ds(46920, 3080)],
                        agg_hbm.at[pl.ds(nbase + 46920, 3080)])


BN = 2000
NGRID = N // BN
BE = 4096
EGRID = E_PAD // BE

_DOT = dict(precision=lax.Precision.HIGHEST, preferred_element_type=jnp.float32)


def _onehot(batch_blk):
    iota = lax.broadcasted_iota(jnp.int32, (1, G), 1)
    return (batch_blk == iota).astype(jnp.float32)


def _mlp2_tc(x, w1, b1, w2, b2):
    h = jnp.maximum(jnp.dot(x, w1, **_DOT) + b1, 0.0)
    return jnp.dot(h, w2, **_DOT) + b2


def _ln_tc(x, g, b):
    mu = jnp.mean(x, axis=-1, keepdims=True)
    var = jnp.mean((x - mu) ** 2, axis=-1, keepdims=True)
    return (x - mu) / jnp.sqrt(var + 1e-5) * g + b


def _full(shape):
    return pl.BlockSpec(shape, lambda i: (0, 0))


def _node_init_body(cfg_ref, b_ref, w1, b1, w2, b2, x_ref, gc_ref, isr_ref):
    i = pl.program_id(0)
    x_ref[...] = _mlp2_tc(cfg_ref[...], w1[...], b1[...], w2[...], b2[...])
    oh = _onehot(b_ref[...])

    @pl.when(i == 0)
    def _():
        gc_ref[...] = jnp.zeros_like(gc_ref)

    gc_ref[...] += jnp.sum(oh, axis=0).reshape(G, 1)

    @pl.when(i == NGRID - 1)
    def _():
        gc = gc_ref[...]
        isr_ref[...] = jnp.where(gc > 0, 1.0 / jnp.sqrt(gc), 0.0)


def _tc_node_init(cfg, batch2, p):
    return pl.pallas_call(
        _node_init_body,
        grid=(NGRID,),
        in_specs=[pl.BlockSpec((BN, 1), lambda i: (i, 0)),
                  pl.BlockSpec((BN, 1), lambda i: (i, 0)),
                  _full((1, H)), _full((1, H)), _full((H, H)), _full((1, H))],
        out_specs=[pl.BlockSpec((BN, H), lambda i: (i, 0)),
                   _full((G, 1)), _full((G, 1))],
        out_shape=[jax.ShapeDtypeStruct((N, H), jnp.float32),
                   jax.ShapeDtypeStruct((G, 1), jnp.float32),
                   jax.ShapeDtypeStruct((G, 1), jnp.float32)],
    )(cfg, batch2, p["w1"], p["b1"].reshape(1, H), p["w2"], p["b2"].reshape(1, H))


def _edge_mlp_body(dr_ref, dc_ref, w1, b1, w2, b2, ea_ref):
    w = w1[...]
    h = jnp.dot(dr_ref[...], w[0:1], **_DOT) + jnp.dot(dc_ref[...], w[1:2], **_DOT)
    h = jnp.maximum(h + b1[...], 0.0)
    ea_ref[...] = jnp.dot(h, w2[...], **_DOT) + b2[...]


def _tc_edge_mlp(dr2, dc2, p):
    return pl.pallas_call(
        _edge_mlp_body,
        grid=(EGRID,),
        in_specs=[pl.BlockSpec((BE, 1), lambda i: (i, 0)),
                  pl.BlockSpec((BE, 1), lambda i: (i, 0)),
                  _full((2, H)), _full((1, H)), _full((H, H)), _full((1, H))],
        out_specs=pl.BlockSpec((BE, H), lambda i: (i, 0)),
        out_shape=jax.ShapeDtypeStruct((E_PAD, H), jnp.float32),
    )(dr2, dc2, p["w1"], p["b1"].reshape(1, H), p["w2"], p["b2"].reshape(1, H))


def _vn_update_body(x_ref, b_ref, vnin, w1, b1, ln1g, ln1b, w2, b2, ln2g, ln2b,
                    vn_ref):
    i = pl.program_id(0)

    @pl.when(i == 0)
    def _():
        vn_ref[...] = jnp.zeros_like(vn_ref)

    oh = _onehot(b_ref[...])
    vn_ref[...] += jnp.dot(oh.T, x_ref[...], **_DOT)

    @pl.when(i == NGRID - 1)
    def _():
        vtemp = vn_ref[...] + vnin[...]
        t = jnp.dot(vtemp, w1[...], **_DOT) + b1[...]
        t = jnp.maximum(_ln_tc(t, ln1g[...], ln1b[...]), 0.0)
        t = jnp.dot(t, w2[...], **_DOT) + b2[...]
        vn_ref[...] = jnp.maximum(_ln_tc(t, ln2g[...], ln2b[...]), 0.0)


def _tc_vn_update(x, batch2, vn_in, vp):
    return pl.pallas_call(
        _vn_update_body,
        grid=(NGRID,),
        in_specs=[pl.BlockSpec((BN, H), lambda i: (i, 0)),
                  pl.BlockSpec((BN, 1), lambda i: (i, 0)),
                  _full((G, H)), _full((H, H)), _full((1, H)), _full((1, H)),
                  _full((1, H)), _full((H, H)), _full((1, H)), _full((1, H)),
                  _full((1, H))],
        out_specs=_full((G, H)),
        out_shape=jax.ShapeDtypeStruct((G, H), jnp.float32),
    )(x, batch2, vn_in, vp["w1"], vp["b1"].reshape(1, H),
      vp["ln1_g"].reshape(1, H), vp["ln1_b"].reshape(1, H),
      vp["w2"], vp["b2"].reshape(1, H),
      vp["ln2_g"].reshape(1, H), vp["ln2_b"].reshape(1, H))


def _node_update_body(x_ref, agg_ref, b_ref, vn, isr, eps, w1, b1, w2, b2,
                      lng, lnb, out_ref):
    x = x_ref[...]
    u = (1.0 + eps[0, 0]) * x + agg_ref[...]
    z = _mlp2_tc(u, w1[...], b1[...], w2[...], b2[...])
    oh = _onehot(b_ref[...])
    z = z * jnp.dot(oh, isr[...], **_DOT)
    z = jnp.maximum(_ln_tc(z, lng[...], lnb[...]), 0.0)
    out_ref[...] = z + x + jnp.dot(oh, vn[...], **_DOT)


def _tc_node_update(x, agg, batch2, vn, isr, eps2, cp, lp):
    return pl.pallas_call(
        _node_update_body,
        grid=(NGRID,),
        in_specs=[pl.BlockSpec((BN, H), lambda i: (i, 0)),
                  pl.BlockSpec((BN, H), lambda i: (i, 0)),
                  pl.BlockSpec((BN, 1), lambda i: (i, 0)),
                  _full((G, H)), _full((G, 1)), _full((1, 1)),
                  _full((H, H)), _full((1, H)), _full((H, H)), _full((1, H)),
                  _full((1, H)), _full((1, H))],
        out_specs=pl.BlockSpec((BN, H), lambda i: (i, 0)),
        out_shape=jax.ShapeDtypeStruct((N, H), jnp.float32),
    )(x, agg, batch2, vn, isr, eps2,
      cp["w1"], cp["b1"].reshape(1, H), cp["w2"], cp["b2"].reshape(1, H),
      lp["g"].reshape(1, H), lp["b"].reshape(1, H))


_NEG = -3.4e38


def _gate_body(x_ref, b_ref, w1, b1, bng, bnb, w2, b2, g_ref, gmax_ref):
    i = pl.program_id(0)
    t = jnp.dot(x_ref[...], w1[...], **_DOT) + b1[...]
    t = t / jnp.sqrt(1.0 + 1e-5) * bng[...] + bnb[...]
    t = jnp.maximum(t, 0.0)
    gn = jnp.dot(t, w2[...], **_DOT) + b2[...]
    g_ref[...] = gn

    @pl.when(i == 0)
    def _():
        gmax_ref[...] = jnp.full_like(gmax_ref, _NEG)

    oh = _onehot(b_ref[...])
    masked = jnp.where(oh > 0, gn, _NEG)
    gmax_ref[...] = jnp.maximum(gmax_ref[...], jnp.max(masked, axis=0).reshape(G, 1))


def _tc_gate(x, batch2, gp):
    return pl.pallas_call(
        _gate_body,
        grid=(NGRID,),
        in_specs=[pl.BlockSpec((BN, H), lambda i: (i, 0)),
                  pl.BlockSpec((BN, 1), lambda i: (i, 0)),
                  _full((H, 2 * H)), _full((1, 2 * H)), _full((1, 2 * H)),
                  _full((1, 2 * H)), _full((2 * H, 1)), _full((1, 1))],
        out_specs=[pl.BlockSpec((BN, 1), lambda i: (i, 0)), _full((G, 1))],
        out_shape=[jax.ShapeDtypeStruct((N, 1), jnp.float32),
                   jax.ShapeDtypeStruct((G, 1), jnp.float32)],
    )(x, batch2, gp["w1"], gp["b1"].reshape(1, 2 * H), gp["bn_g"].reshape(1, 2 * H),
      gp["bn_b"].reshape(1, 2 * H), gp["w2"], gp["b2"].reshape(1, 1))


def _pool_body(x_ref, g_ref, b_ref, gmax, w1, b1, w2, b2,
               s1_ref, s0_ref, out_ref):
    i = pl.program_id(0)

    @pl.when(i == 0)
    def _():
        s1_ref[...] = jnp.zeros_like(s1_ref)
        s0_ref[...] = jnp.zeros_like(s0_ref)
        out_ref[...] = jnp.zeros_like(out_ref)

    oh = _onehot(b_ref[...])
    e = jnp.exp(g_ref[...] - jnp.dot(oh, gmax[...], **_DOT))
    s1_ref[...] += jnp.dot(oh.T, e * x_ref[...], **_DOT)
    s0_ref[...] += jnp.dot(oh.T, e, **_DOT)

    @pl.when(i == NGRID - 1)
    def _():
        s0 = s0_ref[...]
        pool = jnp.where(s0 > 0, s1_ref[...] / jnp.where(s0 > 0, s0, 1.0), 0.0)
        t = _mlp2_tc(pool, w1[...], b1[...], w2[...], b2[...])
        out_ref[...] = 1.0 / (1.0 + jnp.exp(-t))


def _tc_pool_final(x, g, batch2, gmax, fp):
    _, _, out = pl.pallas_call(
        _pool_body,
        grid=(NGRID,),
        in_specs=[pl.BlockSpec((BN, H), lambda i: (i, 0)),
                  pl.BlockSpec((BN, 1), lambda i: (i, 0)),
                  pl.BlockSpec((BN, 1), lambda i: (i, 0)),
                  _full((G, 1)), _full((H, H)), _full((1, H)), _full((H, 1)),
                  _full((1, 1))],
        out_specs=[_full((G, H)), _full((G, 1)), _full((G, 1))],
        out_shape=[jax.ShapeDtypeStruct((G, H), jnp.float32),
                   jax.ShapeDtypeStruct((G, 1), jnp.float32),
                   jax.ShapeDtypeStruct((G, 1), jnp.float32)],
    )(x, g, batch2, gmax, fp["w1"], fp["b1"].reshape(1, H), fp["w2"],
      fp["b2"].reshape(1, 1))
    return out


def kernel(config, edge_index, batch, params):
    pad = E_PAD - E
    row = jnp.concatenate([edge_index[0], jnp.zeros((pad,), jnp.int32)])
    col = jnp.concatenate([edge_index[1], jnp.full((pad,), N, jnp.int32)])
    cfg2 = config.astype(jnp.float32).reshape(N, 1)
    batch2 = batch.reshape(N, 1)
    zeros_deg = jnp.zeros((6400,), jnp.float32)
    ones128 = jnp.ones((CHUNK,), jnp.float32)
    zrows = jnp.zeros((ZROWS, H), jnp.float32)

    deg = _sc_degree(col, zeros_deg, ones128)
    x, _gc, isr = _tc_node_init(cfg2, batch2, params["node_mlp"])
    dr, dc = _sc_gather_deg(row, col, deg)
    ea = _tc_edge_mlp(dr.reshape(E_PAD, 1), dc.reshape(E_PAD, 1),
                      params["edge_mlp"])

    vn = jnp.broadcast_to(params["vn_emb"][0], (G, H))
    for i in range(3):
        agg = _sc_message(row, col, x, ea, zrows)
        vn = _tc_vn_update(x, batch2, vn, params["vn_mlps"][i])
        eps2 = params["convs"][i]["eps"].reshape(1, 1)
        x = _tc_node_update(x, agg, batch2, vn, isr, eps2,
                            params["convs"][i]["nn"], params["lns"][i])

    g, gmax = _tc_gate(x, batch2, params["gate"])
    return _tc_pool_final(x, g, batch2, gmax, params["final_mlp"])


# speedup vs baseline: 4.1253x; 1.0718x over previous
import functools
import jax
import jax.numpy as jnp
from jax import lax
from jax.experimental import pallas as pl
from jax.experimental.pallas import tpu as pltpu
from jax.experimental.pallas import tpu_sc as plsc

N = 100000
E = 1600000
G = 16
H = 32
HALF = N // 2
E_PAD = 1601536
DEG_PAD = 102400
CHUNK = 128
TE_16 = E_PAD // 16
TE_32 = E_PAD // 32
NC_16 = TE_16 // CHUNK
NC_32 = TE_32 // CHUNK
AGG_ROWS = 50176
ZROWS = 3136

_mesh = plsc.VectorSubcoreMesh(core_axis_name="c", subcore_axis_name="s")
_SC_PARAMS = pltpu.CompilerParams(use_tc_tiling_on_sc=False)


@functools.partial(
    pl.kernel, mesh=_mesh, compiler_params=_SC_PARAMS,
    out_type=jax.ShapeDtypeStruct((DEG_PAD,), jnp.float32),
    scratch_types=[
        pltpu.VMEM((CHUNK,), jnp.int32),
        pltpu.VMEM((CHUNK,), jnp.float32),
        pltpu.VMEM_SHARED((DEG_PAD,), jnp.float32),
    ],
)
def _sc_degree(col_hbm, zeros_hbm, ones_hbm, deg_hbm, colv, onev, deg_sh):
    cid = lax.axis_index("c")
    sid = lax.axis_index("s")

    @pl.when(cid == 0)
    def _():
        pltpu.sync_copy(zeros_hbm, deg_sh.at[pl.ds(sid * 6400, 6400)])
        plsc.subcore_barrier()
        pltpu.sync_copy(ones_hbm, onev)

        def body(g, _):
            base = sid * TE_16 + g * CHUNK
            pltpu.sync_copy(col_hbm.at[pl.ds(base, CHUNK)], colv)
            pltpu.sync_copy(onev, deg_sh.at[colv], add=True)
            return _

        lax.fori_loop(0, NC_16, body, None)
        plsc.subcore_barrier()
        pltpu.sync_copy(deg_sh.at[pl.ds(sid * 6400, 6400)],
                        deg_hbm.at[pl.ds(sid * 6400, 6400)])


@functools.partial(
    pl.kernel, mesh=_mesh, compiler_params=_SC_PARAMS,
    out_type=(jax.ShapeDtypeStruct((E_PAD,), jnp.float32),
              jax.ShapeDtypeStruct((E_PAD,), jnp.float32)),
    scratch_types=[
        [pltpu.VMEM((CHUNK,), jnp.int32)] * 2,
        [pltpu.VMEM((CHUNK,), jnp.int32)] * 2,
        [pltpu.VMEM((CHUNK,), jnp.float32)] * 2,
        [pltpu.VMEM((CHUNK,), jnp.float32)] * 2,
        [pltpu.SemaphoreType.DMA] * 2,
        [pltpu.SemaphoreType.DMA] * 2,
        [pltpu.SemaphoreType.DMA] * 2,
        [pltpu.SemaphoreType.DMA] * 2,
        [pltpu.SemaphoreType.DMA] * 2,
        [pltpu.SemaphoreType.DMA] * 2,
    ],
)
def _sc_gather_deg(row_hbm, col_hbm, deg_hbm, dr_hbm, dc_hbm,
                   rv, cv, drv, dcv, sri, sci, srg, scg, srw, scw):
    cid = lax.axis_index("c")
    sid = lax.axis_index("s")
    ebase = (sid * 2 + cid) * TE_32

    def sl(g):
        return pl.ds(ebase + g * CHUNK, CHUNK)

    def start_idx(g, b):
        pltpu.async_copy(row_hbm.at[sl(g)], rv[b], sri[b])
        pltpu.async_copy(col_hbm.at[sl(g)], cv[b], sci[b])

    def start_gather(b):
        pltpu.async_copy(deg_hbm.at[rv[b]], drv[b], srg[b])
        pltpu.async_copy(deg_hbm.at[cv[b]], dcv[b], scg[b])

    def wait_idx(g, b):
        pltpu.make_async_copy(row_hbm.at[sl(g)], rv[b], sri[b]).wait()
        pltpu.make_async_copy(col_hbm.at[sl(g)], cv[b], sci[b]).wait()

    def wait_gather(b):
        pltpu.make_async_copy(deg_hbm.at[rv[b]], drv[b], srg[b]).wait()
        pltpu.make_async_copy(deg_hbm.at[cv[b]], dcv[b], scg[b]).wait()

    def start_write(g, b):
        pltpu.async_copy(drv[b], dr_hbm.at[sl(g)], srw[b])
        pltpu.async_copy(dcv[b], dc_hbm.at[sl(g)], scw[b])

    def wait_write(g, b):
        pltpu.make_async_copy(drv[b], dr_hbm.at[sl(g)], srw[b]).wait()
        pltpu.make_async_copy(dcv[b], dc_hbm.at[sl(g)], scw[b]).wait()

    def emit(g, p, q, pre_idx, drain_w):
        wait_idx(g + 1, q)
        if drain_w:
            wait_write(g - 1, q)
        start_gather(q)
        wait_gather(p)
        if pre_idx:
            start_idx(g + 2, p)
        start_write(g, p)

    start_idx(0, 0)
    start_idx(1, 1)
    wait_idx(0, 0)
    start_gather(0)
    emit(0, 0, 1, True, False)

    def pair_body(i, _):
        g = 1 + i * 2
        emit(g, 1, 0, True, True)
        emit(g + 1, 0, 1, True, True)
        return _

    lax.fori_loop(0, (NC_32 - 3) // 2, pair_body, None)
    emit(NC_32 - 2, 1, 0, False, True)
    wait_gather(0)
    start_write(NC_32 - 1, 0)
    wait_write(NC_32 - 2, 1)
    wait_write(NC_32 - 1, 0)


@functools.partial(
    pl.kernel, mesh=_mesh, compiler_params=_SC_PARAMS,
    out_type=jax.ShapeDtypeStruct((N, H), jnp.float32),
    scratch_types=[
        [pltpu.VMEM((CHUNK,), jnp.int32)] * 2,
        [pltpu.VMEM((CHUNK,), jnp.int32)] * 2,
        [pltpu.VMEM((CHUNK,), jnp.int32)] * 2,
        [pltpu.VMEM((CHUNK, H), jnp.float32)] * 2,
        [pltpu.VMEM((CHUNK, H), jnp.float32)] * 2,
        pltpu.VMEM_SHARED((AGG_ROWS, H), jnp.float32),
        [pltpu.SemaphoreType.DMA] * 2,
        [pltpu.SemaphoreType.DMA] * 2,
        [pltpu.SemaphoreType.DMA] * 2,
        [pltpu.SemaphoreType.DMA] * 2,
    ],
)
def _sc_message(row_hbm, col_hbm, x_hbm, ea_hbm, zrows_hbm, agg_hbm,
                rowv, colv, lcolv, xg, eav, agg_sh, sr, sc_, sx, se):
    cid = lax.axis_index("c")
    sid = lax.axis_index("s")
    nbase = cid * HALF
    ebase = sid * TE_16

    pltpu.sync_copy(zrows_hbm, agg_sh.at[pl.ds(sid * ZROWS, ZROWS)])
    plsc.subcore_barrier()

    def sl(g):
        return pl.ds(ebase + g * CHUNK, CHUNK)

    def start_idx(g, b):
        pltpu.async_copy(row_hbm.at[sl(g)], rowv[b], sr[b])
        pltpu.async_copy(col_hbm.at[sl(g)], colv[b], sc_[b])

    def wait_idx(g, b):
        pltpu.make_async_copy(row_hbm.at[sl(g)], rowv[b], sr[b]).wait()
        pltpu.make_async_copy(col_hbm.at[sl(g)], colv[b], sc_[b]).wait()

    def mk_lcol(b):
        def idx_body(j, _2):
            c16 = colv[b][pl.ds(j * 16, 16)]
            inr = (c16 >= nbase) & (c16 < nbase + HALF)
            lcolv[b][pl.ds(j * 16, 16)] = jnp.where(inr, c16 - nbase, HALF)
            return _2
        lax.fori_loop(0, 8, idx_body, None)

    def start_data(g, b):
        pltpu.async_copy(x_hbm.at[rowv[b]], xg[b], sx[b])
        pltpu.async_copy(ea_hbm.at[sl(g)], eav[b], se[b])

    def wait_data(g, b):
        pltpu.make_async_copy(x_hbm.at[rowv[b]], xg[b], sx[b]).wait()
        pltpu.make_async_copy(ea_hbm.at[sl(g)], eav[b], se[b]).wait()

    def compute_scatter(b):
        def msg_body(r4, _2):
            r = r4 * 4
            for k in range(4):
                a = xg[b][r + k, pl.ds(0, 16)] + eav[b][r + k, pl.ds(0, 16)]
                xg[b][r + k, pl.ds(0, 16)] = jnp.maximum(a, 0.0)
                bb = xg[b][r + k, pl.ds(16, 16)] + eav[b][r + k, pl.ds(16, 16)]
                xg[b][r + k, pl.ds(16, 16)] = jnp.maximum(bb, 0.0)
            return _2
        lax.fori_loop(0, CHUNK // 4, msg_body, None)
        pltpu.sync_copy(xg[b], agg_sh.at[lcolv[b]], add=True)

    def emit(g, p, q, prefetch_idx):
        wait_idx(g + 1, q)
        mk_lcol(q)
        start_data(g + 1, q)
        wait_data(g, p)
        if prefetch_idx:
            start_idx(g + 2, p)
        compute_scatter(p)

    start_idx(0, 0)
    start_idx(1, 1)
    wait_idx(0, 0)
    mk_lcol(0)
    start_data(0, 0)

    def pair_body(i, _):
        g = i * 2
        emit(g, 0, 1, True)
        emit(g + 1, 1, 0, True)
        return _

    lax.fori_loop(0, (NC_16 - 2) // 2, pair_body, None)
    emit(NC_16 - 2, 0, 1, False)
    wait_data(NC_16 - 1, 1)
    compute_scatter(1)
    plsc.subcore_barrier()

    @pl.when(sid < 15)
    def _():
        pltpu.sync_copy(agg_sh.at[pl.ds(sid * 3128, 3128)],
                        agg_hbm.at[pl.ds(nbase + sid * 3128, 3128)])

    @pl.when(sid == 15)
    def _():
        pltpu.sync_copy(agg_sh.at[pl.ds(46920, 3080)],
                        agg_hbm.at[pl.ds(nbase + 46920, 3080)])


BN = 2000
NGRID = N // BN
BE = 4096
EGRID = E_PAD // BE

_DOT = dict(precision=lax.Precision.HIGHEST, preferred_element_type=jnp.float32)


def _onehot(batch_blk):
    iota = lax.broadcasted_iota(jnp.int32, (1, G), 1)
    return (batch_blk == iota).astype(jnp.float32)


def _mlp2_tc(x, w1, b1, w2, b2):
    h = jnp.maximum(jnp.dot(x, w1, **_DOT) + b1, 0.0)
    return jnp.dot(h, w2, **_DOT) + b2


def _ln_tc(x, g, b):
    mu = jnp.mean(x, axis=-1, keepdims=True)
    var = jnp.mean((x - mu) ** 2, axis=-1, keepdims=True)
    return (x - mu) / jnp.sqrt(var + 1e-5) * g + b


def _full(shape):
    return pl.BlockSpec(shape, lambda i: (0, 0))


def _node_init_body(cfg_ref, b_ref, w1, b1, w2, b2, x_ref, gc_ref, isr_ref):
    i = pl.program_id(0)
    x_ref[...] = _mlp2_tc(cfg_ref[...], w1[...], b1[...], w2[...], b2[...])
    oh = _onehot(b_ref[...])

    @pl.when(i == 0)
    def _():
        gc_ref[...] = jnp.zeros_like(gc_ref)

    gc_ref[...] += jnp.sum(oh, axis=0).reshape(G, 1)

    @pl.when(i == NGRID - 1)
    def _():
        gc = gc_ref[...]
        isr_ref[...] = jnp.where(gc > 0, 1.0 / jnp.sqrt(gc), 0.0)


def _tc_node_init(cfg, batch2, p):
    return pl.pallas_call(
        _node_init_body,
        grid=(NGRID,),
        in_specs=[pl.BlockSpec((BN, 1), lambda i: (i, 0)),
                  pl.BlockSpec((BN, 1), lambda i: (i, 0)),
                  _full((1, H)), _full((1, H)), _full((H, H)), _full((1, H))],
        out_specs=[pl.BlockSpec((BN, H), lambda i: (i, 0)),
                   _full((G, 1)), _full((G, 1))],
        out_shape=[jax.ShapeDtypeStruct((N, H), jnp.float32),
                   jax.ShapeDtypeStruct((G, 1), jnp.float32),
                   jax.ShapeDtypeStruct((G, 1), jnp.float32)],
    )(cfg, batch2, p["w1"], p["b1"].reshape(1, H), p["w2"], p["b2"].reshape(1, H))


def _edge_mlp_body(dr_ref, dc_ref, w1, b1, w2, b2, ea_ref):
    w = w1[...]
    h = jnp.dot(dr_ref[...], w[0:1], **_DOT) + jnp.dot(dc_ref[...], w[1:2], **_DOT)
    h = jnp.maximum(h + b1[...], 0.0)
    ea_ref[...] = jnp.dot(h, w2[...], **_DOT) + b2[...]


def _tc_edge_mlp(dr2, dc2, p):
    return pl.pallas_call(
        _edge_mlp_body,
        grid=(EGRID,),
        in_specs=[pl.BlockSpec((BE, 1), lambda i: (i, 0)),
                  pl.BlockSpec((BE, 1), lambda i: (i, 0)),
                  _full((2, H)), _full((1, H)), _full((H, H)), _full((1, H))],
        out_specs=pl.BlockSpec((BE, H), lambda i: (i, 0)),
        out_shape=jax.ShapeDtypeStruct((E_PAD, H), jnp.float32),
    )(dr2, dc2, p["w1"], p["b1"].reshape(1, H), p["w2"], p["b2"].reshape(1, H))


def _vn_update_body(x_ref, b_ref, vnin, w1, b1, ln1g, ln1b, w2, b2, ln2g, ln2b,
                    vn_ref):
    i = pl.program_id(0)

    @pl.when(i == 0)
    def _():
        vn_ref[...] = jnp.zeros_like(vn_ref)

    oh = _onehot(b_ref[...])
    vn_ref[...] += jnp.dot(oh.T, x_ref[...], **_DOT)

    @pl.when(i == NGRID - 1)
    def _():
        vtemp = vn_ref[...] + vnin[...]
        t = jnp.dot(vtemp, w1[...], **_DOT) + b1[...]
        t = jnp.maximum(_ln_tc(t, ln1g[...], ln1b[...]), 0.0)
        t = jnp.dot(t, w2[...], **_DOT) + b2[...]
        vn_ref[...] = jnp.maximum(_ln_tc(t, ln2g[...], ln2b[...]), 0.0)


def _tc_vn_update(x, batch2, vn_in, vp):
    return pl.pallas_call(
        _vn_update_body,
        grid=(NGRID,),
        in_specs=[pl.BlockSpec((BN, H), lambda i: (i, 0)),
                  pl.BlockSpec((BN, 1), lambda i: (i, 0)),
                  _full((G, H)), _full((H, H)), _full((1, H)), _full((1, H)),
                  _full((1, H)), _full((H, H)), _full((1, H)), _full((1, H)),
                  _full((1, H))],
        out_specs=_full((G, H)),
        out_shape=jax.ShapeDtypeStruct((G, H), jnp.float32),
    )(x, batch2, vn_in, vp["w1"], vp["b1"].reshape(1, H),
      vp["ln1_g"].reshape(1, H), vp["ln1_b"].reshape(1, H),
      vp["w2"], vp["b2"].reshape(1, H),
      vp["ln2_g"].reshape(1, H), vp["ln2_b"].reshape(1, H))


def _node_update_body(x_ref, agg_ref, b_ref, vn, isr, eps, w1, b1, w2, b2,
                      lng, lnb, out_ref):
    x = x_ref[...]
    u = (1.0 + eps[0, 0]) * x + agg_ref[...]
    z = _mlp2_tc(u, w1[...], b1[...], w2[...], b2[...])
    oh = _onehot(b_ref[...])
    z = z * jnp.dot(oh, isr[...], **_DOT)
    z = jnp.maximum(_ln_tc(z, lng[...], lnb[...]), 0.0)
    out_ref[...] = z + x + jnp.dot(oh, vn[...], **_DOT)


def _tc_node_update(x, agg, batch2, vn, isr, eps2, cp, lp):
    return pl.pallas_call(
        _node_update_body,
        grid=(NGRID,),
        in_specs=[pl.BlockSpec((BN, H), lambda i: (i, 0)),
                  pl.BlockSpec((BN, H), lambda i: (i, 0)),
                  pl.BlockSpec((BN, 1), lambda i: (i, 0)),
                  _full((G, H)), _full((G, 1)), _full((1, 1)),
                  _full((H, H)), _full((1, H)), _full((H, H)), _full((1, H)),
                  _full((1, H)), _full((1, H))],
        out_specs=pl.BlockSpec((BN, H), lambda i: (i, 0)),
        out_shape=jax.ShapeDtypeStruct((N, H), jnp.float32),
    )(x, agg, batch2, vn, isr, eps2,
      cp["w1"], cp["b1"].reshape(1, H), cp["w2"], cp["b2"].reshape(1, H),
      lp["g"].reshape(1, H), lp["b"].reshape(1, H))


_NEG = -3.4e38


def _gate_body(x_ref, b_ref, w1, b1, bng, bnb, w2, b2, g_ref, gmax_ref):
    i = pl.program_id(0)
    t = jnp.dot(x_ref[...], w1[...], **_DOT) + b1[...]
    t = t / jnp.sqrt(1.0 + 1e-5) * bng[...] + bnb[...]
    t = jnp.maximum(t, 0.0)
    gn = jnp.dot(t, w2[...], **_DOT) + b2[...]
    g_ref[...] = gn

    @pl.when(i == 0)
    def _():
        gmax_ref[...] = jnp.full_like(gmax_ref, _NEG)

    oh = _onehot(b_ref[...])
    masked = jnp.where(oh > 0, gn, _NEG)
    gmax_ref[...] = jnp.maximum(gmax_ref[...], jnp.max(masked, axis=0).reshape(G, 1))


def _tc_gate(x, batch2, gp):
    return pl.pallas_call(
        _gate_body,
        grid=(NGRID,),
        in_specs=[pl.BlockSpec((BN, H), lambda i: (i, 0)),
                  pl.BlockSpec((BN, 1), lambda i: (i, 0)),
                  _full((H, 2 * H)), _full((1, 2 * H)), _full((1, 2 * H)),
                  _full((1, 2 * H)), _full((2 * H, 1)), _full((1, 1))],
        out_specs=[pl.BlockSpec((BN, 1), lambda i: (i, 0)), _full((G, 1))],
        out_shape=[jax.ShapeDtypeStruct((N, 1), jnp.float32),
                   jax.ShapeDtypeStruct((G, 1), jnp.float32)],
    )(x, batch2, gp["w1"], gp["b1"].reshape(1, 2 * H), gp["bn_g"].reshape(1, 2 * H),
      gp["bn_b"].reshape(1, 2 * H), gp["w2"], gp["b2"].reshape(1, 1))


def _pool_body(x_ref, g_ref, b_ref, gmax, w1, b1, w2, b2,
               s1_ref, s0_ref, out_ref):
    i = pl.program_id(0)

    @pl.when(i == 0)
    def _():
        s1_ref[...] = jnp.zeros_like(s1_ref)
        s0_ref[...] = jnp.zeros_like(s0_ref)
        out_ref[...] = jnp.zeros_like(out_ref)

    oh = _onehot(b_ref[...])
    e = jnp.exp(g_ref[...] - jnp.dot(oh, gmax[...], **_DOT))
    s1_ref[...] += jnp.dot(oh.T, e * x_ref[...], **_DOT)
    s0_ref[...] += jnp.dot(oh.T, e, **_DOT)

    @pl.when(i == NGRID - 1)
    def _():
        s0 = s0_ref[...]
        pool = jnp.where(s0 > 0, s1_ref[...] / jnp.where(s0 > 0, s0, 1.0), 0.0)
        t = _mlp2_tc(pool, w1[...], b1[...], w2[...], b2[...])
        out_ref[...] = 1.0 / (1.0 + jnp.exp(-t))


def _tc_pool_final(x, g, batch2, gmax, fp):
    _, _, out = pl.pallas_call(
        _pool_body,
        grid=(NGRID,),
        in_specs=[pl.BlockSpec((BN, H), lambda i: (i, 0)),
                  pl.BlockSpec((BN, 1), lambda i: (i, 0)),
                  pl.BlockSpec((BN, 1), lambda i: (i, 0)),
                  _full((G, 1)), _full((H, H)), _full((1, H)), _full((H, 1)),
                  _full((1, 1))],
        out_specs=[_full((G, H)), _full((G, 1)), _full((G, 1))],
        out_shape=[jax.ShapeDtypeStruct((G, H), jnp.float32),
                   jax.ShapeDtypeStruct((G, 1), jnp.float32),
                   jax.ShapeDtypeStruct((G, 1), jnp.float32)],
    )(x, g, batch2, gmax, fp["w1"], fp["b1"].reshape(1, H), fp["w2"],
      fp["b2"].reshape(1, 1))
    return out


def kernel(config, edge_index, batch, params):
    pad = E_PAD - E
    row = jnp.concatenate([edge_index[0], jnp.zeros((pad,), jnp.int32)])
    col = jnp.concatenate([edge_index[1], jnp.full((pad,), N, jnp.int32)])
    cfg2 = config.astype(jnp.float32).reshape(N, 1)
    batch2 = batch.reshape(N, 1)
    zeros_deg = jnp.zeros((6400,), jnp.float32)
    ones128 = jnp.ones((CHUNK,), jnp.float32)
    zrows = jnp.zeros((ZROWS, H), jnp.float32)

    deg = _sc_degree(col, zeros_deg, ones128)
    x, _gc, isr = _tc_node_init(cfg2, batch2, params["node_mlp"])
    dr, dc = _sc_gather_deg(row, col, deg)
    ea = _tc_edge_mlp(dr.reshape(E_PAD, 1), dc.reshape(E_PAD, 1),
                      params["edge_mlp"])

    vn = jnp.broadcast_to(params["vn_emb"][0], (G, H))
    for i in range(3):
        agg = _sc_message(row, col, x, ea, zrows)
        vn = _tc_vn_update(x, batch2, vn, params["vn_mlps"][i])
        eps2 = params["convs"][i]["eps"].reshape(1, 1)
        x = _tc_node_update(x, agg, batch2, vn, isr, eps2,
                            params["convs"][i]["nn"], params["lns"][i])

    g, gmax = _tc_gate(x, batch2, params["gate"])
    return _tc_pool_final(x, g, batch2, gmax, params["final_mlp"])
